# baseline (speedup 1.0000x reference)
import jax
import jax.numpy as jnp
from jax import lax
from jax.experimental import pallas as pl
from jax.experimental.pallas import tpu as pltpu
from jax.experimental.pallas import tpu_sc as plsc

NUM_NODES = 2708
NUM_EDGES = 500000
D = 128
NP = 2816
NC = 2
NS = 16
CR = 352
NCHUNK = NP // CR
PASSES = NCHUNK // NC
CHUNK_W = CR * NP
SLAB = CHUNK_W // NS
B = 4096
NB = 8
E_REAL_PS = NUM_EDGES // NS
E_PROC_PS = NB * B
E_STG_PS = E_PROC_PS + 2 * B


def _hist_body(src_hbm, dst_hbm, zeros_hbm, a_hbm,
               sb_a, db_a, sb_b, db_b, ib_a, vb_a, ib_b, vb_b, chunk,
               sem_ea, sem_eb, sem_sa, sem_sb):
    c = lax.axis_index("c")
    s = lax.axis_index("s")
    ebase = s * E_STG_PS

    def start_e(blk, sb, db, sem):
        off = ebase + blk * B
        pltpu.async_copy(src_hbm.at[pl.ds(off, B)], sb, sem)
        pltpu.async_copy(dst_hbm.at[pl.ds(off, B)], db, sem)

    def drain_e(sb, db, sem):
        pltpu.make_async_copy(src_hbm.at[pl.ds(0, B)], sb, sem).wait()
        pltpu.make_async_copy(dst_hbm.at[pl.ds(0, B)], db, sem).wait()

    def compute(sb, db, ib, vb, base):
        def slice_body(i, carry):
            sv = sb[pl.ds(i * 16, 16)]
            dv = db[pl.ds(i * 16, 16)]
            t = dv * NP + sv - base
            m = (t >= 0) & (t < CHUNK_W)
            ib[pl.ds(i * 16, 16)] = jnp.where(m, t, 0)
            vb[pl.ds(i * 16, 16)] = jnp.where(m, 1.0, 0.0)
            return carry
        lax.fori_loop(0, B // 16, slice_body, 0)

    def start_s(ib, vb, sem):
        pltpu.async_copy(vb, chunk.at[ib], sem, add=True)

    def drain_s(ib, vb, sem):
        pltpu.make_async_copy(vb, chunk.at[ib], sem).wait()

    for p in range(PASSES):
        k = p * NC + c
        base = k * CHUNK_W
        pltpu.sync_copy(zeros_hbm, chunk.at[pl.ds(s * SLAB, SLAB)])
        plsc.subcore_barrier()

        start_e(0, sb_a, db_a, sem_ea)
        start_e(1, sb_b, db_b, sem_eb)
        drain_e(sb_a, db_a, sem_ea)
        compute(sb_a, db_a, ib_a, vb_a, base)
        start_s(ib_a, vb_a, sem_sa)
        start_e(2, sb_a, db_a, sem_ea)
        drain_e(sb_b, db_b, sem_eb)
        compute(sb_b, db_b, ib_b, vb_b, base)
        start_s(ib_b, vb_b, sem_sb)
        start_e(3, sb_b, db_b, sem_eb)

        def pair(q, carry):
            drain_e(sb_a, db_a, sem_ea)
            drain_s(ib_a, vb_a, sem_sa)
            compute(sb_a, db_a, ib_a, vb_a, base)
            start_s(ib_a, vb_a, sem_sa)
            start_e(2 * q + 2, sb_a, db_a, sem_ea)
            drain_e(sb_b, db_b, sem_eb)
            drain_s(ib_b, vb_b, sem_sb)
            compute(sb_b, db_b, ib_b, vb_b, base)
            start_s(ib_b, vb_b, sem_sb)
            start_e(2 * q + 3, sb_b, db_b, sem_eb)
            return carry

        lax.fori_loop(1, NB // 2, pair, 0)
        drain_e(sb_a, db_a, sem_ea)
        drain_e(sb_b, db_b, sem_eb)
        drain_s(ib_a, vb_a, sem_sa)
        drain_s(ib_b, vb_b, sem_sb)
        plsc.subcore_barrier()
        pltpu.sync_copy(chunk.at[pl.ds(s * SLAB, SLAB)],
                        a_hbm.at[pl.ds(base + s * SLAB, SLAB)])


def _build_counts(src, dst, zeros):
    mesh = plsc.VectorSubcoreMesh(core_axis_name="c", subcore_axis_name="s")
    return pl.kernel(
        _hist_body,
        out_type=jax.ShapeDtypeStruct((NP * NP,), jnp.float32),
        mesh=mesh,
        scratch_types=[
            pltpu.VMEM((B,), jnp.int32),
            pltpu.VMEM((B,), jnp.int32),
            pltpu.VMEM((B,), jnp.int32),
            pltpu.VMEM((B,), jnp.int32),
            pltpu.VMEM((B,), jnp.int32),
            pltpu.VMEM((B,), jnp.float32),
            pltpu.VMEM((B,), jnp.int32),
            pltpu.VMEM((B,), jnp.float32),
            pltpu.VMEM_SHARED((CHUNK_W,), jnp.float32),
            pltpu.SemaphoreType.DMA,
            pltpu.SemaphoreType.DMA,
            pltpu.SemaphoreType.DMA,
            pltpu.SemaphoreType.DMA,
        ],
    )(src, dst, zeros)


def _mm_body(a_ref, x_ref, w_ref, b_ref, o_ref):
    a = a_ref[...]
    ax = jnp.dot(a, x_ref[...], preferred_element_type=jnp.float32)
    h = lax.dot_general(ax, w_ref[...], (((1,), (1,)), ((), ())),
                        preferred_element_type=jnp.float32)
    deg = jnp.sum(a, axis=1, keepdims=True)
    o_ref[...] = h + deg * b_ref[...]


def _gcn_matmul(a2d, x_pad, weight, bias2d):
    BM = 256
    grid = (NP // BM,)
    return pl.pallas_call(
        _mm_body,
        grid=grid,
        in_specs=[
            pl.BlockSpec((BM, NP), lambda i: (i, 0)),
            pl.BlockSpec((NP, D), lambda i: (0, 0)),
            pl.BlockSpec((D, D), lambda i: (0, 0)),
            pl.BlockSpec((1, D), lambda i: (0, 0)),
        ],
        out_specs=pl.BlockSpec((BM, D), lambda i: (i, 0)),
        out_shape=jax.ShapeDtypeStruct((NP, D), jnp.float32),
    )(a2d, x_pad, weight, bias2d)


def kernel(x, edge_index, weight, bias):
    src = edge_index[0].astype(jnp.int32)
    dst = edge_index[1].astype(jnp.int32)
    pad = E_STG_PS - E_REAL_PS
    src16 = src.reshape(NS, E_REAL_PS)
    dst16 = dst.reshape(NS, E_REAL_PS)
    src_p = jnp.concatenate(
        [src16, jnp.zeros((NS, pad), jnp.int32)], axis=1).reshape(-1)
    dst_p = jnp.concatenate(
        [dst16, jnp.full((NS, pad), NP * 2, jnp.int32)], axis=1).reshape(-1)

    zeros = jnp.zeros((SLAB,), jnp.float32)
    a_flat = _build_counts(src_p, dst_p, zeros)
    a2d = a_flat.reshape(NP, NP)

    x_pad = jnp.zeros((NP, D), jnp.float32).at[:NUM_NODES].set(x)
    out = _gcn_matmul(a2d, x_pad, weight, bias.reshape(1, D))
    return out[:NUM_NODES]


# speedup vs baseline: 11.5837x; 11.5837x over previous
import jax
import jax.numpy as jnp
from jax import lax
from jax.experimental import pallas as pl
from jax.experimental.pallas import tpu as pltpu
from jax.experimental.pallas import tpu_sc as plsc

NUM_NODES = 2708
NUM_EDGES = 500000
D = 128
NP = 2816
NC = 2
NS = 16
CR = 704
NCHUNK = NP // CR
PASSES = NCHUNK // NC
CHUNK_W = CR * NP
SLAB = CHUNK_W // NS
B = 512
NB = 62
E_REAL_PS = NUM_EDGES // NS
E_PROC_PS = NB * B
E_STG_PS = E_PROC_PS + 2 * B


def _hist_body(src_hbm, dst_hbm, zeros_hbm, a_hbm,
               sb_a, db_a, sb_b, db_b, ib_a, vb_a, ib_b, vb_b, chunk,
               sem_ea, sem_eb, sem_sa, sem_sb):
    c = lax.axis_index("c")
    s = lax.axis_index("s")
    ebase = s * E_STG_PS

    def start_e(blk, sb, db, sem):
        off = ebase + blk * B
        pltpu.async_copy(src_hbm.at[pl.ds(off, B)], sb, sem)
        pltpu.async_copy(dst_hbm.at[pl.ds(off, B)], db, sem)

    def drain_e(sb, db, sem):
        pltpu.make_async_copy(src_hbm.at[pl.ds(0, B)], sb, sem).wait()
        pltpu.make_async_copy(dst_hbm.at[pl.ds(0, B)], db, sem).wait()

    def compute(sb, db, ib, vb, base):
        def slice_body(i, carry):
            sv = sb[pl.ds(i * 16, 16)]
            dv = db[pl.ds(i * 16, 16)]
            t = dv * NP + sv - base
            m = (t >= 0) & (t < CHUNK_W)
            ib[pl.ds(i * 16, 16)] = jnp.where(m, t, t & 0xFFFFF)
            vb[pl.ds(i * 16, 16)] = jnp.where(m, 1.0, 0.0)
            return carry
        lax.fori_loop(0, B // 16, slice_body, 0)

    def start_s(ib, vb, sem):
        pltpu.async_copy(vb, chunk.at[ib], sem, add=True)

    def drain_s(ib, vb, sem):
        pltpu.make_async_copy(vb, chunk.at[ib], sem).wait()

    for p in range(PASSES):
        k = p * NC + c
        base = k * CHUNK_W
        pltpu.sync_copy(zeros_hbm, chunk.at[pl.ds(s * SLAB, SLAB)])
        plsc.subcore_barrier()

        start_e(0, sb_a, db_a, sem_ea)
        start_e(1, sb_b, db_b, sem_eb)
        drain_e(sb_a, db_a, sem_ea)
        compute(sb_a, db_a, ib_a, vb_a, base)
        start_s(ib_a, vb_a, sem_sa)
        start_e(2, sb_a, db_a, sem_ea)
        drain_e(sb_b, db_b, sem_eb)
        compute(sb_b, db_b, ib_b, vb_b, base)
        start_s(ib_b, vb_b, sem_sb)
        start_e(3, sb_b, db_b, sem_eb)

        def pair(q, carry):
            drain_e(sb_a, db_a, sem_ea)
            drain_s(ib_a, vb_a, sem_sa)
            compute(sb_a, db_a, ib_a, vb_a, base)
            start_s(ib_a, vb_a, sem_sa)
            start_e(2 * q + 2, sb_a, db_a, sem_ea)
            drain_e(sb_b, db_b, sem_eb)
            drain_s(ib_b, vb_b, sem_sb)
            compute(sb_b, db_b, ib_b, vb_b, base)
            start_s(ib_b, vb_b, sem_sb)
            start_e(2 * q + 3, sb_b, db_b, sem_eb)
            return carry

        lax.fori_loop(1, NB // 2, pair, 0)
        drain_e(sb_a, db_a, sem_ea)
        drain_e(sb_b, db_b, sem_eb)
        drain_s(ib_a, vb_a, sem_sa)
        drain_s(ib_b, vb_b, sem_sb)
        plsc.subcore_barrier()
        pltpu.sync_copy(chunk.at[pl.ds(s * SLAB, SLAB)],
                        a_hbm.at[pl.ds(base + s * SLAB, SLAB)])


def _build_counts(src, dst, zeros):
    mesh = plsc.VectorSubcoreMesh(core_axis_name="c", subcore_axis_name="s")
    return pl.kernel(
        _hist_body,
        out_type=jax.ShapeDtypeStruct((NP * NP,), jnp.float32),
        mesh=mesh,
        scratch_types=[
            pltpu.VMEM((B,), jnp.int32),
            pltpu.VMEM((B,), jnp.int32),
            pltpu.VMEM((B,), jnp.int32),
            pltpu.VMEM((B,), jnp.int32),
            pltpu.VMEM((B,), jnp.int32),
            pltpu.VMEM((B,), jnp.float32),
            pltpu.VMEM((B,), jnp.int32),
            pltpu.VMEM((B,), jnp.float32),
            pltpu.VMEM_SHARED((CHUNK_W,), jnp.float32),
            pltpu.SemaphoreType.DMA,
            pltpu.SemaphoreType.DMA,
            pltpu.SemaphoreType.DMA,
            pltpu.SemaphoreType.DMA,
        ],
    )(src, dst, zeros)


def _mm_body(a_ref, x_ref, w_ref, b_ref, o_ref):
    a = a_ref[...]
    ax = jnp.dot(a, x_ref[...], preferred_element_type=jnp.float32)
    h = lax.dot_general(ax, w_ref[...], (((1,), (1,)), ((), ())),
                        preferred_element_type=jnp.float32)
    deg = jnp.sum(a, axis=1, keepdims=True)
    o_ref[...] = h + deg * b_ref[...]


def _gcn_matmul(a2d, x_pad, weight, bias2d):
    BM = 256
    grid = (NP // BM,)
    return pl.pallas_call(
        _mm_body,
        grid=grid,
        in_specs=[
            pl.BlockSpec((BM, NP), lambda i: (i, 0)),
            pl.BlockSpec((NP, D), lambda i: (0, 0)),
            pl.BlockSpec((D, D), lambda i: (0, 0)),
            pl.BlockSpec((1, D), lambda i: (0, 0)),
        ],
        out_specs=pl.BlockSpec((BM, D), lambda i: (i, 0)),
        out_shape=jax.ShapeDtypeStruct((NP, D), jnp.float32),
    )(a2d, x_pad, weight, bias2d)


def kernel(x, edge_index, weight, bias):
    src = edge_index[0].astype(jnp.int32)
    dst = edge_index[1].astype(jnp.int32)
    pad = E_STG_PS - E_REAL_PS
    src16 = src.reshape(NS, E_REAL_PS)
    dst16 = dst.reshape(NS, E_REAL_PS)
    src_p = jnp.concatenate(
        [src16, jnp.zeros((NS, pad), jnp.int32)], axis=1).reshape(-1)
    dst_p = jnp.concatenate(
        [dst16, jnp.full((NS, pad), NP * 2, jnp.int32)], axis=1).reshape(-1)

    zeros = jnp.zeros((SLAB,), jnp.float32)
    a_flat = _build_counts(src_p, dst_p, zeros)
    a2d = a_flat.reshape(NP, NP)

    x_pad = jnp.zeros((NP, D), jnp.float32).at[:NUM_NODES].set(x)
    out = _gcn_matmul(a2d, x_pad, weight, bias.reshape(1, D))
    return out[:NUM_NODES]
